# baseline (speedup 1.0000x reference)
import functools

import jax
import jax.numpy as jnp
from jax import lax
from jax.experimental import pallas as pl
from jax.experimental.pallas import tpu as pltpu
from jax.experimental.pallas import tpu_sc as plsc

N_F = 26
V = 100000
E = 32
B = 16384

NC = 2
NS = 16
L = 16
FPC = N_F // NC

CV = 512
NSL = V // CV
TAIL = V - NSL * CV
NT_ALL = FPC * NSL
SR = V // 4

BPT = B // NS
HB = 256
NSTEP = FPC * (BPT // HB)
CHUNK = 128


def _convert(c, s, tab_hbm, tail_hbm, scr_hbm, chunk, tbuf, gsem, wsem):
    nt = (NT_ALL - s + NS - 1) // NS

    def params(t):
        fl = t // NSL
        sl = t - fl * NSL
        return 13 * c + fl, sl * CV

    def start_in(t):
        f, v0 = params(t)
        p = lax.rem(t // NS, 2)
        for r in range(4):
            pltpu.make_async_copy(
                tab_hbm.at[f, pl.ds(8 * r, 8), pl.ds(v0, CV)],
                chunk.at[p, pl.ds(8 * r, 8), :],
                gsem,
            ).start()

    def wait_in(p):
        pltpu.make_async_copy(
            tab_hbm.at[0, :, pl.ds(0, CV)], chunk.at[p], gsem
        ).wait()

    def transpose(p):
        def row(gr, carry):
            for j in range(8):
                evec = jax.lax.iota(jnp.int32, L) + 16 * (j % 2)
                vvec = jnp.zeros((L,), jnp.int32) + (4 * gr + j // 2)
                tbuf[p, gr, pl.ds(16 * j, L)] = plsc.load_gather(
                    chunk.at[p], [evec, vvec]
                )
            return carry

        lax.fori_loop(0, CV // 4, row, 0)

    def w_start(t, p):
        f, v0 = params(t)
        pltpu.make_async_copy(
            tbuf.at[p],
            scr_hbm.at[pl.ds(pl.multiple_of(f * SR + v0 // 4, 8), CV // 4), :],
            wsem,
        ).start()

    def w_wait():
        pltpu.make_async_copy(
            tbuf.at[0], scr_hbm.at[pl.ds(0, CV // 4), :], wsem
        ).wait()

    start_in(s)

    def step(i, carry):
        t = s + NS * i
        p = lax.rem(i, 2)

        wait_in(p)

        @pl.when(i + 1 < nt)
        def _():
            start_in(t + NS)

        @pl.when(i >= 2)
        def _():
            w_wait()

        transpose(p)
        w_start(t, p)
        return carry

    lax.fori_loop(0, nt, step, 0)
    w_wait()
    w_wait()

    @pl.when(s < FPC)
    def _():
        f = 13 * c + s
        pltpu.sync_copy(tail_hbm.at[f], tbuf.at[0, pl.ds(0, TAIL // 4), :])
        pltpu.sync_copy(
            tbuf.at[0, pl.ds(0, TAIL // 4), :],
            scr_hbm.at[
                pl.ds(pl.multiple_of(f * SR + (NSL * CV) // 4, 8), TAIL // 4), :
            ],
        )

def _gather(c, s, idx_hbm, scr_hbm, out_hbm, idx_st, rows_v, t_v, gv, ov,
            gsem, wsem):
    b0 = s * BPT
    pltpu.sync_copy(idx_hbm.at[:, pl.ds(pl.multiple_of(b0, 128), BPT)], idx_st)

    def prep(st, p):
        f = 13 * c + st // 4
        h = lax.rem(st, 4)
        for k in range(HB // L):
            iv = idx_st[f, pl.ds(h * HB + k * L, L)]
            gv[p, pl.ds(k * L, L)] = iv >> 2
            ov[p, pl.ds(k * L, L)] = (iv & 3) << 5

    def g_start(p):
        for cc in range(HB // CHUNK):
            pltpu.make_async_copy(
                scr_hbm.at[gv.at[p, pl.ds(cc * CHUNK, CHUNK)]],
                rows_v.at[p, pl.ds(cc * CHUNK, CHUNK), :],
                gsem,
            ).start()

    def g_wait(p):
        pltpu.make_async_copy(
            scr_hbm.at[pl.ds(0, HB)], rows_v.at[p], gsem
        ).wait()

    def transpose(p):
        def erow(e, carry):
            for k in range(HB // L):
                bvec = jax.lax.iota(jnp.int32, L) + k * L
                evec = ov[p, pl.ds(k * L, L)] + e
                t_v[p, e, pl.ds(k * L, L)] = plsc.load_gather(
                    rows_v.at[p], [bvec, evec]
                )
            return carry

        lax.fori_loop(0, E, erow, 0)

    def w_desc(st, p):
        f = 13 * c + st // 4
        h = lax.rem(st, 4)
        return pltpu.make_async_copy(
            t_v.at[p],
            out_hbm.at[pl.ds(pl.multiple_of(f * E, 8), E), pl.ds(pl.multiple_of(b0 + h * HB, 128), HB)],
            wsem,
        )

    prep(0, 0)
    g_start(0)
    g_wait(0)
    prep(1, 1)
    g_start(1)
    transpose(0)
    w_desc(0, 0).start()

    def step(st, carry):
        p = lax.rem(st, 2)
        q = 1 - p
        g_wait(p)

        @pl.when(st + 1 < NSTEP)
        def _():
            prep(st + 1, q)
            g_start(q)

        w_desc(st - 1, q).wait()
        transpose(p)
        w_desc(st, p).start()
        return carry

    lax.fori_loop(1, NSTEP, step, 0)
    w_desc(NSTEP - 1, (NSTEP - 1) % 2).wait()


def _body(idx_hbm, tab_hbm, tail_hbm, out_hbm, scr_hbm, gsem, wsem):
    c = lax.axis_index("c")
    s = lax.axis_index("s")

    def phase1(chunk, tbuf):
        _convert(c, s, tab_hbm, tail_hbm, scr_hbm, chunk, tbuf, gsem, wsem)

    pl.run_scoped(
        phase1,
        pltpu.VMEM((2, E, CV), jnp.float32),
        pltpu.VMEM((2, CV // 4, 128), jnp.float32),
    )

    plsc.subcore_barrier()

    def phase2(idx_st, rows_v, t_v, gv, ov):
        _gather(c, s, idx_hbm, scr_hbm, out_hbm, idx_st, rows_v, t_v,
                gv, ov, gsem, wsem)

    pl.run_scoped(
        phase2,
        pltpu.VMEM((N_F, BPT), jnp.int32),
        pltpu.VMEM((2, HB, 128), jnp.float32),
        pltpu.VMEM((2, E, HB), jnp.float32),
        pltpu.VMEM((2, HB), jnp.int32),
        pltpu.VMEM((2, HB), jnp.int32),
    )


@functools.partial(
    pl.kernel,
    out_type=(
        jax.ShapeDtypeStruct((N_F * E, B), jnp.float32),
        jax.ShapeDtypeStruct((N_F * SR, 128), jnp.float32),
    ),
    mesh=plsc.VectorSubcoreMesh(core_axis_name="c", subcore_axis_name="s"),
    compiler_params=pltpu.CompilerParams(
        use_tc_tiling_on_sc=True, needs_layout_passes=False
    ),
    scratch_types=[
        pltpu.SemaphoreType.DMA,
        pltpu.SemaphoreType.DMA,
    ],
)
def _gather_kernel(idx_hbm, tab_hbm, tail_hbm, out_hbm, scr_hbm, gsem, wsem):
    _body(idx_hbm, tab_hbm, tail_hbm, out_hbm, scr_hbm, gsem, wsem)


def kernel(families, tables):
    fam = families.astype(jnp.int32)
    offs = (jnp.arange(N_F, dtype=jnp.int32) * V)[:, None]
    idx2 = fam + offs
    tabT = jnp.transpose(tables, (0, 2, 1))
    tails = tables[:, NSL * CV :, :].reshape(N_F, TAIL // 4, 128)
    outT, _ = _gather_kernel(idx2, tabT, tails)
    return outT.T


# speedup vs baseline: 2.6063x; 2.6063x over previous
import functools

import jax
import jax.numpy as jnp
from jax import lax
from jax.experimental import pallas as pl
from jax.experimental.pallas import tpu as pltpu
from jax.experimental.pallas import tpu_sc as plsc

N_F = 26
V = 100000
E = 32
B = 16384

NC = 2
NS = 16
L = 16
HOUT = B // 2


def _body(idx_hbm, tab_hbm, out_hbm, idx_v, plane, orow, gsem):
    e = lax.axis_index("s") * NC + lax.axis_index("c")

    def field(f, carry):
        pltpu.sync_copy(tab_hbm.at[f, e, :], plane)
        pltpu.sync_copy(idx_hbm.at[f, :], idx_v)

        def half(h):
            def chunk(k, carry):
                iv = idx_v[pl.ds(h * HOUT + k * L, L)]
                orow[pl.ds(k * L, L)] = plsc.load_gather(plane, [iv])
                return carry

            lax.fori_loop(0, HOUT // L, chunk, 0)
            pltpu.sync_copy(
                orow, out_hbm.at[f * E + e, pl.ds(h * HOUT, HOUT)]
            )

        half(0)
        half(1)
        return carry

    lax.fori_loop(0, N_F, field, 0)


@functools.partial(
    pl.kernel,
    out_type=jax.ShapeDtypeStruct((N_F * E, B), jnp.float32),
    mesh=plsc.VectorSubcoreMesh(core_axis_name="c", subcore_axis_name="s"),
    compiler_params=pltpu.CompilerParams(
        use_tc_tiling_on_sc=False, needs_layout_passes=False
    ),
    scratch_types=[
        pltpu.VMEM((B,), jnp.int32),
        pltpu.VMEM((V,), jnp.float32),
        pltpu.VMEM((HOUT,), jnp.float32),
        pltpu.SemaphoreType.DMA,
    ],
)
def _gather_kernel(idx_hbm, tab_hbm, out_hbm, idx_v, plane, orow, gsem):
    _body(idx_hbm, tab_hbm, out_hbm, idx_v, plane, orow, gsem)


def kernel(families, tables):
    fam = families.astype(jnp.int32)
    tabT = jnp.transpose(tables, (0, 2, 1))
    outT = _gather_kernel(fam, tabT)
    return outT.T


# speedup vs baseline: 2.6429x; 1.0141x over previous
import functools

import jax
import jax.numpy as jnp
from jax import lax
from jax.experimental import pallas as pl
from jax.experimental.pallas import tpu as pltpu
from jax.experimental.pallas import tpu_sc as plsc

N_F = 26
V = 100000
E = 32
B = 16384

NC = 2
NS = 16
L = 16
HOUT = B // 4


def _body(idx_hbm, tab_hbm, out_hbm, idx_v, plane, orow, gsem, wsem):
    e = lax.axis_index("s") * NC + lax.axis_index("c")

    def p_start(f):
        pltpu.make_async_copy(tab_hbm.at[f, e, :], plane, gsem).start()

    def p_wait():
        pltpu.make_async_copy(tab_hbm.at[0, 0, :], plane, gsem).wait()

    def w_desc(f, q):
        return pltpu.make_async_copy(
            orow.at[lax.rem(q, 2)],
            out_hbm.at[f * E + e, pl.ds(q * HOUT, HOUT)],
            wsem,
        )

    p_start(0)

    def field(f, carry):
        pltpu.sync_copy(idx_hbm.at[f, :], idx_v)
        p_wait()

        def w_wait():
            pltpu.make_async_copy(
                orow.at[0], out_hbm.at[0, pl.ds(0, HOUT)], wsem
            ).wait()

        def quarter(q):
            def chunk(k, carry):
                iv = idx_v[pl.ds(q * HOUT + k * L, L)]
                orow[q % 2, pl.ds(k * L, L)] = plsc.load_gather(
                    plane, [iv]
                )
                return carry

            lax.fori_loop(0, HOUT // L, chunk, 0)

            if q >= 2:
                w_wait()
            else:
                @pl.when(f > 0)
                def _():
                    w_wait()

            w_desc(f, q).start()

        for q in range(4):
            quarter(q)

        @pl.when(f + 1 < N_F)
        def _():
            p_start(f + 1)

        return carry

    lax.fori_loop(0, N_F, field, 0)
    for _ in range(2):
        pltpu.make_async_copy(
            orow.at[0], out_hbm.at[0, pl.ds(0, HOUT)], wsem
        ).wait()


@functools.partial(
    pl.kernel,
    out_type=jax.ShapeDtypeStruct((N_F * E, B), jnp.float32),
    mesh=plsc.VectorSubcoreMesh(core_axis_name="c", subcore_axis_name="s"),
    compiler_params=pltpu.CompilerParams(
        use_tc_tiling_on_sc=False, needs_layout_passes=False
    ),
    scratch_types=[
        pltpu.VMEM((B,), jnp.int32),
        pltpu.VMEM((V,), jnp.float32),
        pltpu.VMEM((2, HOUT), jnp.float32),
        pltpu.SemaphoreType.DMA,
        pltpu.SemaphoreType.DMA,
    ],
)
def _gather_kernel(idx_hbm, tab_hbm, out_hbm, idx_v, plane, orow, gsem, wsem):
    _body(idx_hbm, tab_hbm, out_hbm, idx_v, plane, orow, gsem, wsem)


def kernel(families, tables):
    fam = families.astype(jnp.int32)
    tabT = jnp.transpose(tables, (0, 2, 1))
    outT = _gather_kernel(fam, tabT)
    return outT.T


# speedup vs baseline: 2.8027x; 1.0604x over previous
import functools

import jax
import jax.numpy as jnp
from jax import lax
from jax.experimental import pallas as pl
from jax.experimental.pallas import tpu as pltpu
from jax.experimental.pallas import tpu_sc as plsc

N_F = 26
V = 100000
E = 32
B = 16384

NC = 2
NS = 16
L = 16
HOUT = B // 4


def _body(idx_hbm, tab_hbm, out_hbm, idx_v, plane, orow, gsem, wsem):
    e = lax.axis_index("s") * NC + lax.axis_index("c")

    def p_start(f):
        pltpu.make_async_copy(tab_hbm.at[f, e, :], plane, gsem).start()

    def p_wait():
        pltpu.make_async_copy(tab_hbm.at[0, 0, :], plane, gsem).wait()

    def w_desc(f, q):
        return pltpu.make_async_copy(
            orow.at[lax.rem(q, 2)],
            out_hbm.at[f * E + e, pl.ds(q * HOUT, HOUT)],
            wsem,
        )

    p_start(0)

    def field(f, carry):
        pltpu.sync_copy(idx_hbm.at[f, :], idx_v)
        p_wait()

        def w_wait():
            pltpu.make_async_copy(
                orow.at[0], out_hbm.at[0, pl.ds(0, HOUT)], wsem
            ).wait()

        def quarter(q):
            def chunk(k, carry):
                for u in range(8):
                    o = k * 8 * L + u * L
                    iv = idx_v[pl.ds(q * HOUT + o, L)]
                    orow[q % 2, pl.ds(o, L)] = plsc.load_gather(
                        plane, [iv]
                    )
                return carry

            lax.fori_loop(0, HOUT // L // 8, chunk, 0)

            if q >= 2:
                w_wait()
            else:
                @pl.when(f > 0)
                def _():
                    w_wait()

            w_desc(f, q).start()

        for q in range(4):
            quarter(q)

        @pl.when(f + 1 < N_F)
        def _():
            p_start(f + 1)

        return carry

    lax.fori_loop(0, N_F, field, 0)
    for _ in range(2):
        pltpu.make_async_copy(
            orow.at[0], out_hbm.at[0, pl.ds(0, HOUT)], wsem
        ).wait()


@functools.partial(
    pl.kernel,
    out_type=jax.ShapeDtypeStruct((N_F * E, B), jnp.float32),
    mesh=plsc.VectorSubcoreMesh(core_axis_name="c", subcore_axis_name="s"),
    compiler_params=pltpu.CompilerParams(
        use_tc_tiling_on_sc=False, needs_layout_passes=False
    ),
    scratch_types=[
        pltpu.VMEM((B,), jnp.int32),
        pltpu.VMEM((V,), jnp.float32),
        pltpu.VMEM((2, HOUT), jnp.float32),
        pltpu.SemaphoreType.DMA,
        pltpu.SemaphoreType.DMA,
    ],
)
def _gather_kernel(idx_hbm, tab_hbm, out_hbm, idx_v, plane, orow, gsem, wsem):
    _body(idx_hbm, tab_hbm, out_hbm, idx_v, plane, orow, gsem, wsem)


def kernel(families, tables):
    fam = families.astype(jnp.int32)
    tabT = jnp.transpose(tables, (0, 2, 1))
    outT = _gather_kernel(fam, tabT)
    return outT.T
